# Initial kernel scaffold; baseline (speedup 1.0000x reference)
import jax
import jax.numpy as jnp
from jax.experimental import pallas as pl


def kernel(edge_index, edge_attrs, embedding_weight):
    raise NotImplementedError("write your pallas kernel here")


# speedup vs baseline: 7.6568x; 7.6568x over previous
import functools

import jax
import jax.numpy as jnp
from jax import lax
from jax.experimental import pallas as pl
from jax.experimental.pallas import tpu as pltpu
from jax.experimental.pallas import tpu_sc as plsc

N = 100000
D = 32
E = 1600000
N_LAYERS = 3

HALF = 50000
T = 3200
S_ROWS = 51200
TRASH = 50048
PAD = S_ROWS - HALF
P = 2 * S_ROWS

C = 128
NCH = E // C
BASE_CH = NCH // 16
EXTRA = NCH - 16 * BASE_CH

B = 400
RB = HALF // B
PB = S_ROWS // B

_mesh = plsc.VectorSubcoreMesh(core_axis_name="c", subcore_axis_name="s")


def _edge_range(s):
    start = s * BASE_CH + jnp.minimum(s, EXTRA)
    count = BASE_CH + (s < EXTRA).astype(jnp.int32)
    return start, start + count


def _remap_dst(dstraw, idxbuf, base):
    for r in range(C // 16):
        v = dstraw[pl.ds(r * 16, 16)]
        local = v - base
        ok = (local >= 0) & (local < HALF)
        idxbuf[0, pl.ds(r * 16, 16)] = jnp.where(ok, local, TRASH)


@functools.partial(
    pl.kernel,
    mesh=_mesh,
    compiler_params=pltpu.CompilerParams(use_tc_tiling_on_sc=False),
    out_type=jax.ShapeDtypeStruct((P,), jnp.float32),
    scratch_types=[
        pltpu.VMEM_SHARED((S_ROWS,), jnp.float32),
        pltpu.VMEM((C,), jnp.int32),
        pltpu.VMEM((1, C), jnp.int32),
        pltpu.VMEM((C,), jnp.float32),
    ],
)
def _degree_sc(dst_hbm, deg_hbm, deg_sp, dstraw, idxbuf, vals):
    c = lax.axis_index("c")
    s = lax.axis_index("s")
    base = c * HALF

    for r in range(C // 16):
        vals[pl.ds(r * 16, 16)] = jnp.zeros((16,), jnp.float32)

    def _zero(j, carry):
        pltpu.sync_copy(vals, deg_sp.at[pl.ds(s * T + j * C, C)])
        return carry
    lax.fori_loop(0, T // C, _zero, 0)

    for r in range(C // 16):
        vals[pl.ds(r * 16, 16)] = jnp.ones((16,), jnp.float32)
    plsc.subcore_barrier()

    lo, hi = _edge_range(s)

    def _deg(ch, carry):
        pltpu.sync_copy(dst_hbm.at[pl.ds(ch * C, C)], dstraw)
        _remap_dst(dstraw, idxbuf, base)
        pltpu.sync_copy(vals, deg_sp.at[idxbuf.at[0]], add=True)
        return carry
    lax.fori_loop(lo, hi, _deg, 0)
    plsc.subcore_barrier()

    pltpu.sync_copy(deg_sp.at[pl.ds(s * T, T)],
                    deg_hbm.at[pl.ds(c * S_ROWS + s * T, T)])


@functools.partial(
    pl.kernel,
    mesh=_mesh,
    compiler_params=pltpu.CompilerParams(use_tc_tiling_on_sc=False),
    out_type=jax.ShapeDtypeStruct((P, D), jnp.float32),
    scratch_types=[
        pltpu.VMEM_SHARED((S_ROWS, D), jnp.float32),
        pltpu.VMEM((C,), jnp.int32),
        pltpu.VMEM((C,), jnp.int32),
        pltpu.VMEM((1, C), jnp.int32),
        pltpu.VMEM((C, D), jnp.float32),
    ],
)
def _scatter_sc(src_hbm, dst_hbm, y_hbm, s_out_hbm,
                s_sp, srcbuf, dstraw, idxbuf, rows):
    c = lax.axis_index("c")
    s = lax.axis_index("s")
    base = c * HALF

    def _zrow(r, carry):
        rows[r, pl.ds(0, 16)] = jnp.zeros((16,), jnp.float32)
        rows[r, pl.ds(16, 16)] = jnp.zeros((16,), jnp.float32)
        return carry
    lax.fori_loop(0, C, _zrow, 0)

    def _zero(j, carry):
        pltpu.sync_copy(rows, s_sp.at[pl.ds(s * T + j * C, C)])
        return carry
    lax.fori_loop(0, T // C, _zero, 0)
    plsc.subcore_barrier()

    lo, hi = _edge_range(s)

    def _edge(ch, carry):
        pltpu.sync_copy(src_hbm.at[pl.ds(ch * C, C)], srcbuf)
        pltpu.sync_copy(dst_hbm.at[pl.ds(ch * C, C)], dstraw)
        for r in range(C // 16):
            v = srcbuf[pl.ds(r * 16, 16)]
            srcbuf[pl.ds(r * 16, 16)] = jnp.where(v >= HALF, v + PAD, v)
        _remap_dst(dstraw, idxbuf, base)
        pltpu.sync_copy(y_hbm.at[srcbuf], rows)
        pltpu.sync_copy(rows, s_sp.at[idxbuf.at[0]], add=True)
        return carry
    lax.fori_loop(lo, hi, _edge, 0)
    plsc.subcore_barrier()

    def _out(j, carry):
        pltpu.sync_copy(s_sp.at[pl.ds(s * T + j * C, C)],
                        s_out_hbm.at[pl.ds(c * S_ROWS + s * T + j * C, C)])
        return carry
    lax.fori_loop(0, T // C, _out, 0)


def _setup_tc_body(deg_ref, emb_ref, dis32_ref, y0_ref, acc0_ref):
    deg = deg_ref[...]
    d = jnp.where(deg > 0.5, lax.rsqrt(deg), 0.0)
    d32 = jnp.broadcast_to(d, (B, D))
    e = emb_ref[...]
    dis32_ref[...] = d32
    y0_ref[...] = d32 * e
    acc0_ref[...] = 0.25 * e


def _emb_map(i):
    half, loc = i // PB, i % PB
    return (jnp.where(loc < RB, half * RB + loc, 0), 0)


_setup_tc = pl.pallas_call(
    _setup_tc_body,
    grid=(2 * PB,),
    in_specs=[
        pl.BlockSpec((B, 1), lambda i: (i, 0)),
        pl.BlockSpec((B, D), _emb_map),
    ],
    out_specs=[
        pl.BlockSpec((B, D), lambda i: (i, 0)),
        pl.BlockSpec((B, D), lambda i: (i, 0)),
        pl.BlockSpec((B, D), lambda i: (i, 0)),
    ],
    out_shape=[
        jax.ShapeDtypeStruct((P, D), jnp.float32),
        jax.ShapeDtypeStruct((P, D), jnp.float32),
        jax.ShapeDtypeStruct((P, D), jnp.float32),
    ],
)


def _post_tc_body(s_ref, dis32_ref, acc_ref, y_ref, acco_ref):
    d32 = dis32_ref[...]
    t = d32 * s_ref[...]
    y_ref[...] = d32 * t
    acco_ref[...] = acc_ref[...] + 0.25 * t


_post_tc = pl.pallas_call(
    _post_tc_body,
    grid=(2 * PB,),
    in_specs=[pl.BlockSpec((B, D), lambda i: (i, 0))] * 3,
    out_specs=[pl.BlockSpec((B, D), lambda i: (i, 0))] * 2,
    out_shape=[
        jax.ShapeDtypeStruct((P, D), jnp.float32),
        jax.ShapeDtypeStruct((P, D), jnp.float32),
    ],
)


def _final_tc_body(s_ref, dis32_ref, acc_ref, out_ref):
    t = dis32_ref[...] * s_ref[...]
    out_ref[...] = acc_ref[...] + 0.25 * t


def _pad_map(i):
    return (i + (PAD // B) * (i // RB), 0)


_final_tc = pl.pallas_call(
    _final_tc_body,
    grid=(2 * RB,),
    in_specs=[pl.BlockSpec((B, D), _pad_map)] * 3,
    out_specs=pl.BlockSpec((B, D), lambda i: (i, 0)),
    out_shape=jax.ShapeDtypeStruct((N, D), jnp.float32),
)


def kernel(edge_index, edge_attrs, embedding_weight):
    del edge_attrs
    src = edge_index[0]
    dst = edge_index[1]
    deg = _degree_sc(dst).reshape(P, 1)
    dis32, y, acc = _setup_tc(deg, embedding_weight)
    for layer in range(N_LAYERS):
        s_pad = _scatter_sc(src, dst, y)
        if layer < N_LAYERS - 1:
            y, acc = _post_tc(s_pad, dis32, acc)
        else:
            out = _final_tc(s_pad, dis32, acc)
    return (embedding_weight, out)


# speedup vs baseline: 33.0142x; 4.3118x over previous
import functools

import jax
import jax.numpy as jnp
from jax import lax
from jax.experimental import pallas as pl
from jax.experimental.pallas import tpu as pltpu
from jax.experimental.pallas import tpu_sc as plsc

N = 100000
D = 32
E = 1600000
N_LAYERS = 3

HALF = 50000
T = 3200
S_ROWS = 51200
TRASH = 50048
PAD = S_ROWS - HALF
P = 2 * S_ROWS

C = 128
G = 4
GE = G * C
CPT = 784
NCH = 16 * CPT
E_PAD = NCH * C
NG = CPT // G
EPT = CPT * C
CAP = EPT + 4 * GE
CROWS = CAP // C
CBUF = 4 * GE + 16
DB = 200
LG = 3
LGE = LG * C

_mesh = plsc.VectorSubcoreMesh(core_axis_name="c", subcore_axis_name="s")
_sc_params = pltpu.CompilerParams(use_tc_tiling_on_sc=False,
                                  needs_layout_passes=False)


def _rsqrt16(x):
    i = lax.bitcast_convert_type(x, jnp.int32)
    i = jnp.int32(0x5F3759DF) - lax.shift_right_logical(i, 1)
    y = lax.bitcast_convert_type(i, jnp.float32)
    for _ in range(3):
        y = y * (1.5 - 0.5 * x * y * y)
    return jnp.where(x > 0.5, y, 0.0)


@functools.partial(
    pl.kernel,
    mesh=_mesh,
    compiler_params=_sc_params,
    out_type=[
        jax.ShapeDtypeStruct((P,), jnp.float32),
        jax.ShapeDtypeStruct((P, D), jnp.float32),
        jax.ShapeDtypeStruct((P * D,), jnp.float32),
        jax.ShapeDtypeStruct((2, 16, CAP), jnp.int32),
        jax.ShapeDtypeStruct((2, 16, CROWS, C), jnp.int32),
        jax.ShapeDtypeStruct((2, 16, 8), jnp.int32),
    ],
    scratch_types=[
        pltpu.VMEM_SHARED((S_ROWS,), jnp.float32),
        pltpu.VMEM((2, GE), jnp.int32),
        pltpu.VMEM((2, GE), jnp.int32),
        pltpu.VMEM((2, G, C), jnp.int32),
        pltpu.VMEM((C,), jnp.float32),
        pltpu.VMEM((CBUF,), jnp.int32),
        pltpu.VMEM((CBUF,), jnp.int32),
        pltpu.VMEM((G, C), jnp.int32),
        pltpu.VMEM((16,), jnp.int32),
        pltpu.VMEM((T,), jnp.float32),
        pltpu.VMEM((DB * D,), jnp.float32),
        pltpu.VMEM((DB, D), jnp.float32),
        pltpu.VMEM((DB * D,), jnp.float32),
        pltpu.SemaphoreType.DMA((2,)),
        pltpu.SemaphoreType.DMA((G,)),
    ],
)
def _prep_sc(src_hbm, dst_hbm, embf_hbm,
             dis_hbm, y0_hbm, acc0_hbm, csrc_hbm, cdst_hbm, cnt_hbm,
             deg_sp, raws, rawd, idxblk, vals, cs, cd, cd2, cntv,
             disv, embbuf, ybuf, accbuf, rsem, ssem):
    c = lax.axis_index("c")
    s = lax.axis_index("s")
    base = c * HALF
    ebase = s * EPT

    for r in range(C // 16):
        vals[pl.ds(r * 16, 16)] = jnp.zeros((16,), jnp.float32)

    def _zero(j, carry):
        pltpu.sync_copy(vals, deg_sp.at[pl.ds(s * T + j * C, C)])
        return carry
    lax.fori_loop(0, T // C, _zero, 0)

    for r in range(C // 16):
        vals[pl.ds(r * 16, 16)] = jnp.ones((16,), jnp.float32)
    plsc.subcore_barrier()

    def _fire_raw(g, p):
        pltpu.async_copy(src_hbm.at[pl.ds(ebase + g * GE, GE)],
                         raws.at[p], rsem.at[p])
        pltpu.async_copy(dst_hbm.at[pl.ds(ebase + g * GE, GE)],
                         rawd.at[p], rsem.at[p])

    def _wait_raw(p):
        pltpu.make_async_copy(src_hbm.at[pl.ds(0, GE)],
                              raws.at[p], rsem.at[p]).wait()
        pltpu.make_async_copy(dst_hbm.at[pl.ds(0, GE)],
                              rawd.at[p], rsem.at[p]).wait()

    def _flush(nf):
        pltpu.sync_copy(cs.at[pl.ds(0, GE)],
                        csrc_hbm.at[c, s, pl.ds(nf * GE, GE)])
        for k in range(G):
            for r in range(C // 16):
                cd2[k, pl.ds(r * 16, 16)] = cd[pl.ds(k * C + r * 16, 16)]
        pltpu.sync_copy(cd2, cdst_hbm.at[c, s, pl.ds(nf * G, G)])
        for k in range(GE // 16):
            cs[pl.ds(k * 16, 16)] = cs[pl.ds(GE + k * 16, 16)]
            cd[pl.ds(k * 16, 16)] = cd[pl.ds(GE + k * 16, 16)]

    def _group(g, p, first, off, nf):
        del g
        _wait_raw(p)
        for i in range(G):
            if not first:
                pltpu.make_async_copy(
                    vals, deg_sp.at[idxblk.at[p, i]], ssem.at[i]).wait()
            for r in range(C // 16):
                sl = pl.ds(i * C + r * 16, 16)
                v = raws[p, sl]
                w = rawd[p, sl]
                local = w - base
                ok = (local >= 0) & (local < HALF)
                idxblk[p, i, pl.ds(r * 16, 16)] = jnp.where(
                    ok, local, TRASH + s)
                srcp = jnp.where(v >= HALF, v + PAD, v)
                plsc.store_compressed(cs.at[pl.ds(off, 16)], srcp, mask=ok)
                plsc.store_compressed(cd.at[pl.ds(off, 16)], local, mask=ok)
                off = off + jnp.sum(ok.astype(jnp.int32))
            pltpu.async_copy(vals, deg_sp.at[idxblk.at[p, i]], ssem.at[i],
                             add=True)
        fl = off >= GE

        @pl.when(fl)
        def _():
            _flush(nf)
        off = jnp.where(fl, off - GE, off)
        nf = nf + fl.astype(jnp.int32)
        return off, nf

    _fire_raw(0, 0)
    _fire_raw(1, 1)
    off, nf = _group(0, 0, True, jnp.int32(0), jnp.int32(0))
    _fire_raw(2, 0)
    off, nf = _group(1, 1, False, off, nf)

    def _pair(t, carry):
        off, nf = carry
        _fire_raw(2 * t + 1, 1)
        off, nf = _group(2 * t, 0, False, off, nf)

        @pl.when(t < NG // 2 - 1)
        def _():
            _fire_raw(2 * t + 2, 0)
        off, nf = _group(2 * t + 1, 1, False, off, nf)
        return off, nf
    off, nf = lax.fori_loop(1, NG // 2, _pair, (off, nf))

    cnt = nf * GE + off
    for k in range(2 * GE // 16):
        cs[pl.ds(off + k * 16, 16)] = jnp.zeros((16,), jnp.int32)
        cd[pl.ds(off + k * 16, 16)] = jnp.full((16,), TRASH, jnp.int32) + s
    for b in range(4):
        _flush(nf + b)
    cntv[pl.ds(0, 16)] = jnp.full((16,), 1, jnp.int32) * cnt
    pltpu.sync_copy(cntv.at[pl.ds(0, 8)], cnt_hbm.at[c, s])

    for i in range(G):
        pltpu.make_async_copy(vals, deg_sp.at[idxblk.at[0, i]],
                              ssem.at[i]).wait()
    plsc.subcore_barrier()

    pltpu.sync_copy(deg_sp.at[pl.ds(s * T, T)], disv)

    def _dis(i, carry):
        disv[pl.ds(i * 16, 16)] = _rsqrt16(disv[pl.ds(i * 16, 16)])
        return carry
    lax.fori_loop(0, T // 16, _dis, 0)
    pltpu.sync_copy(disv, dis_hbm.at[pl.ds(c * S_ROWS + s * T, T)])

    for k in range(T // DB):
        row_l = s * T + k * DB

        @pl.when(row_l < HALF)
        def _():
            g0u = base + row_l
            pltpu.sync_copy(embf_hbm.at[pl.ds(g0u * D, DB * D)], embbuf)

            def _node(n, carry):
                dd = plsc.load_gather(
                    disv, [jnp.full((16,), k * DB + n, jnp.int32)])
                e0 = embbuf[pl.ds(n * D, 16)]
                e1 = embbuf[pl.ds(n * D + 16, 16)]
                ybuf[n, pl.ds(0, 16)] = dd * e0
                ybuf[n, pl.ds(16, 16)] = dd * e1
                accbuf[pl.ds(n * D, 16)] = 0.25 * e0
                accbuf[pl.ds(n * D + 16, 16)] = 0.25 * e1
                return carry
            lax.fori_loop(0, DB, _node, 0)
            gp = c * S_ROWS + row_l
            pltpu.sync_copy(ybuf, y0_hbm.at[pl.ds(gp, DB)])
            pltpu.sync_copy(accbuf, acc0_hbm.at[pl.ds(gp * D, DB * D)])


def _make_layer(final):
    if final:
        outs = jax.ShapeDtypeStruct((N, D), jnp.float32)
    else:
        outs = [
            jax.ShapeDtypeStruct((P, D), jnp.float32),
            jax.ShapeDtypeStruct((P * D,), jnp.float32),
        ]

    @functools.partial(
        pl.kernel,
        mesh=_mesh,
        compiler_params=_sc_params,
        out_type=outs,
        scratch_types=[
            pltpu.VMEM_SHARED((S_ROWS, D), jnp.float32),
            pltpu.VMEM((2, LGE), jnp.int32),
            pltpu.VMEM((2, LG, C), jnp.int32),
            pltpu.VMEM((LG, C, D), jnp.float32),
            pltpu.VMEM((16,), jnp.int32),
            pltpu.VMEM((DB,), jnp.float32),
            pltpu.VMEM((DB, D), jnp.float32),
            pltpu.VMEM((DB * D,), jnp.float32),
            pltpu.SemaphoreType.DMA((2,)),
            pltpu.SemaphoreType.DMA((LG,)),
            pltpu.SemaphoreType.DMA((LG,)),
        ],
    )
    def _layer(csrc_hbm, cdst_hbm, cnt_hbm, dis_hbm, y_hbm, acc_hbm,
               *refs):
        if final:
            (out_hbm, s_sp, raws, rawd, rows, cntv, disb, sblk, accb,
             rsem, gsem, ssem) = refs
        else:
            (yo_hbm, acco_hbm, s_sp, raws, rawd, rows, cntv, disb, sblk,
             accb, rsem, gsem, ssem) = refs
        c = lax.axis_index("c")
        s = lax.axis_index("s")

        pltpu.sync_copy(cnt_hbm.at[c, s], cntv.at[pl.ds(0, 8)])
        cnt = cntv[pl.ds(0, 16)][0]
        npairs = jnp.maximum(1, (cnt + 2 * LGE - 1) // (2 * LGE))

        def _zrow(r, carry):
            rows[0, r, pl.ds(0, 16)] = jnp.zeros((16,), jnp.float32)
            rows[0, r, pl.ds(16, 16)] = jnp.zeros((16,), jnp.float32)
            return carry
        lax.fori_loop(0, C, _zrow, 0)

        def _zero(j, carry):
            pltpu.sync_copy(rows.at[0], s_sp.at[pl.ds(s * T + j * C, C)])
            return carry
        lax.fori_loop(0, T // C, _zero, 0)
        plsc.subcore_barrier()

        def _fire_raw(g, p):
            pltpu.async_copy(csrc_hbm.at[c, s, pl.ds(g * LGE, LGE)],
                             raws.at[p], rsem.at[p])
            pltpu.async_copy(cdst_hbm.at[c, s, pl.ds(g * LG, LG)],
                             rawd.at[p], rsem.at[p])

        def _wait_raw(p):
            pltpu.make_async_copy(csrc_hbm.at[c, s, pl.ds(0, LGE)],
                                  raws.at[p], rsem.at[p]).wait()
            pltpu.make_async_copy(cdst_hbm.at[c, s, pl.ds(0, LG)],
                                  rawd.at[p], rsem.at[p]).wait()

        def _pair(t, carry):
            _wait_raw(0)
            for i in range(LG):
                @pl.when(t > 0)
                def _():
                    pltpu.make_async_copy(
                        rows.at[i], s_sp.at[rawd.at[0, i]],
                        ssem.at[i]).wait()
                pltpu.async_copy(y_hbm.at[raws.at[0, pl.ds(i * C, C)]],
                                 rows.at[i], gsem.at[i])
            _fire_raw(2 * t + 1, 1)
            for i in range(LG):
                pltpu.make_async_copy(y_hbm.at[raws.at[0, pl.ds(i * C, C)]],
                                      rows.at[i], gsem.at[i]).wait()
                pltpu.async_copy(rows.at[i], s_sp.at[rawd.at[0, i]],
                                 ssem.at[i], add=True)
            _wait_raw(1)
            for i in range(LG):
                pltpu.make_async_copy(
                    rows.at[i], s_sp.at[rawd.at[1, i]], ssem.at[i]).wait()
                pltpu.async_copy(y_hbm.at[raws.at[1, pl.ds(i * C, C)]],
                                 rows.at[i], gsem.at[i])

            @pl.when(t + 1 < npairs)
            def _():
                _fire_raw(2 * t + 2, 0)
            for i in range(LG):
                pltpu.make_async_copy(y_hbm.at[raws.at[1, pl.ds(i * C, C)]],
                                      rows.at[i], gsem.at[i]).wait()
                pltpu.async_copy(rows.at[i], s_sp.at[rawd.at[1, i]],
                                 ssem.at[i], add=True)
            return carry

        _fire_raw(0, 0)
        lax.fori_loop(0, npairs, _pair, 0)

        for i in range(LG):
            pltpu.make_async_copy(rows.at[i], s_sp.at[rawd.at[1, i]],
                                  ssem.at[i]).wait()
        plsc.subcore_barrier()

        for k in range(T // DB):
            row_l = s * T + k * DB
            valid = row_l < HALF

            def _drain():
                gp = c * S_ROWS + row_l
                pltpu.sync_copy(s_sp.at[pl.ds(row_l, DB)], sblk)
                pltpu.sync_copy(acc_hbm.at[pl.ds(gp * D, DB * D)], accb)
                pltpu.sync_copy(dis_hbm.at[pl.ds(gp, DB)], disb)

                def _node(n, carry):
                    dd = plsc.load_gather(
                        disb, [jnp.full((16,), n, jnp.int32)])
                    t0 = dd * sblk[n, pl.ds(0, 16)]
                    t1 = dd * sblk[n, pl.ds(16, 16)]
                    a0 = accb[pl.ds(n * D, 16)] + 0.25 * t0
                    a1 = accb[pl.ds(n * D + 16, 16)] + 0.25 * t1
                    if final:
                        sblk[n, pl.ds(0, 16)] = a0
                        sblk[n, pl.ds(16, 16)] = a1
                    else:
                        sblk[n, pl.ds(0, 16)] = dd * t0
                        sblk[n, pl.ds(16, 16)] = dd * t1
                        accb[pl.ds(n * D, 16)] = a0
                        accb[pl.ds(n * D + 16, 16)] = a1
                    return carry
                lax.fori_loop(0, DB, _node, 0)
                if final:
                    g0u = c * HALF + row_l
                    pltpu.sync_copy(sblk, out_hbm.at[pl.ds(g0u, DB)])
                else:
                    pltpu.sync_copy(sblk, yo_hbm.at[pl.ds(gp, DB)])
                    pltpu.sync_copy(accb, acco_hbm.at[pl.ds(gp * D, DB * D)])

            if final:
                pl.when(valid)(_drain)
            else:
                pl.when(valid)(_drain)
    return _layer


_layer_mid = _make_layer(False)
_layer_fin = _make_layer(True)


def kernel(edge_index, edge_attrs, embedding_weight):
    del edge_attrs
    src = jnp.concatenate(
        [edge_index[0], jnp.zeros((E_PAD - E,), jnp.int32)])
    dst = jnp.concatenate(
        [edge_index[1], jnp.full((E_PAD - E,), -1, jnp.int32)])
    embf = embedding_weight.reshape(-1)
    dis, y, acc, csrc, cdst, cnts = _prep_sc(src, dst, embf)
    for _ in range(N_LAYERS - 1):
        y, acc = _layer_mid(csrc, cdst, cnts, dis, y, acc)
    out = _layer_fin(csrc, cdst, cnts, dis, y, acc)
    return (embedding_weight, out)
